# baseline (speedup 1.0000x reference)
import jax
import jax.numpy as jnp
from jax.experimental import pallas as pl


def _copy_kernel(k_ref, v_ref, k_out_ref, v_out_ref):
    k_out_ref[...] = k_ref[...]
    v_out_ref[...] = v_ref[...]


def kernel(k_val, v_val, k_cache, v_cache):
    del k_cache, v_cache
    b, s, h, d = k_val.shape
    blk = 16
    spec = pl.BlockSpec((blk, s, h, d), lambda i: (i, 0, 0, 0))
    k_out, v_out = pl.pallas_call(
        _copy_kernel,
        grid=(b // blk,),
        in_specs=[spec, spec],
        out_specs=[spec, spec],
        out_shape=[
            jax.ShapeDtypeStruct((b, s, h, d), k_val.dtype),
            jax.ShapeDtypeStruct((b, s, h, d), v_val.dtype),
        ],
    )(k_val, v_val)
    return (k_out, v_out)


# speedup vs baseline: 1.1493x; 1.1493x over previous
import jax
import jax.numpy as jnp
from jax.experimental import pallas as pl


def _copy_kernel(k_ref, v_ref, k_out_ref, v_out_ref):
    k_out_ref[...] = k_ref[...]
    v_out_ref[...] = v_ref[...]


def kernel(k_val, v_val, k_cache, v_cache):
    del k_cache, v_cache
    b, s, h, d = k_val.shape
    blk = 8
    spec = pl.BlockSpec((blk, s, h, d), lambda i: (i, 0, 0, 0))
    k_out, v_out = pl.pallas_call(
        _copy_kernel,
        grid=(b // blk,),
        in_specs=[spec, spec],
        out_specs=[spec, spec],
        out_shape=[
            jax.ShapeDtypeStruct((b, s, h, d), k_val.dtype),
            jax.ShapeDtypeStruct((b, s, h, d), v_val.dtype),
        ],
    )(k_val, v_val)
    return (k_out, v_out)
